# baseline (speedup 1.0000x reference)
import functools

import jax
import jax.numpy as jnp
from jax import lax
from jax.experimental import pallas as pl
from jax.experimental.pallas import tpu as pltpu
from jax.experimental.pallas import tpu_sc as plsc

N = 50000
C = 10
E = 3200000
EPS = 1e-15

N_PAD = 51200
BN = 51200
BE = 4096
CHUNKS = 3
CH = 1066672
NFULL = 260
NB16 = BE // 16


def _edge_pass(xn_t, src, dst):
    mesh = plsc.VectorSubcoreMesh(core_axis_name="c", subcore_axis_name="s")

    @functools.partial(
        pl.kernel,
        out_type=jax.ShapeDtypeStruct((C, CHUNKS, N_PAD), jnp.float32),
        mesh=mesh,
        compiler_params=pltpu.CompilerParams(needs_layout_passes=False),
        scratch_types=[
            pltpu.VMEM((N_PAD,), jnp.float32),
            pltpu.VMEM((N_PAD,), jnp.float32),
            pltpu.VMEM((BE,), jnp.int32),
            pltpu.VMEM((BE,), jnp.int32),
            pltpu.VMEM((BE,), jnp.int32),
            pltpu.VMEM((BE,), jnp.int32),
            pltpu.SemaphoreType.DMA,
            pltpu.SemaphoreType.DMA,
        ],
    )
    def k(xn_hbm, src_hbm, dst_hbm, out_hbm, table_v, acc_v, sb0, sb1,
          db0, db1, sem0, sem1):
        w = lax.axis_index("s") * 2 + lax.axis_index("c")

        @pl.when(w < C * CHUNKS)
        def _():
            cls = w // CHUNKS
            chunk = w % CHUNKS
            start = chunk * CH
            cnt = jnp.minimum(CH, E - start)
            tail16 = (cnt - NFULL * BE) // 16

            sbufs, dbufs, sems = (sb0, sb1), (db0, db1), (sem0, sem1)

            tcopy = pltpu.make_async_copy(xn_hbm.at[cls], table_v, sem0)
            tcopy.start()

            @plsc.parallel_loop(0, N_PAD // 16, unroll=4)
            def _(i):
                acc_v[pl.ds(i * 16, 16)] = jnp.zeros((16,), jnp.float32)

            tcopy.wait()

            def start_blk(b, slot):
                off = start + jnp.minimum(b * BE, cnt - BE)
                pltpu.async_copy(src_hbm.at[pl.ds(off, BE)], sbufs[slot],
                                 sems[slot])
                pltpu.async_copy(dst_hbm.at[pl.ds(off, BE)], dbufs[slot],
                                 sems[slot])

            def wait_blk(slot):
                pltpu.make_async_copy(src_hbm.at[pl.ds(0, BE)], sbufs[slot],
                                      sems[slot]).wait()
                pltpu.make_async_copy(dst_hbm.at[pl.ds(0, BE)], dbufs[slot],
                                      sems[slot]).wait()

            def group16(sb, db, i):
                s = sb[pl.ds(i * 16, 16)]
                d = db[pl.ds(i * 16, 16)]
                vals = plsc.load_gather(table_v, [s])
                plsc.addupdate_scatter(acc_v, [d], vals)

            start_blk(0, 0)
            start_blk(1, 1)

            def pair(g, _):
                for slot in (0, 1):
                    b = 2 * g + slot
                    wait_blk(slot)

                    @pl.when(b + 2 <= NFULL)
                    def _():
                        start_blk(b + 2, slot)

                    @plsc.parallel_loop(0, NB16, unroll=8)
                    def _(i):
                        group16(sbufs[slot], dbufs[slot], i)
                return 0

            lax.fori_loop(0, NFULL // 2, pair, 0)

            wait_blk(0)

            @plsc.parallel_loop(NB16 - tail16, NB16, unroll=2)
            def _(i):
                group16(sb0, db0, i)

            pltpu.sync_copy(acc_v, out_hbm.at[cls, chunk])

    return k(xn_t, src, dst)


def _tc_pre(x):
    def body(x_ref, o_ref):
        xb = x_ref[...]
        s = jnp.sum(xb * xb, axis=1, keepdims=True)
        xn = xb * (1.0 / (jnp.sqrt(s) + EPS))
        o_ref[:, :N] = xn.T
        o_ref[:, N:] = jnp.zeros((C, N_PAD - N), jnp.float32)

    return pl.pallas_call(
        body,
        grid=(1,),
        in_specs=[pl.BlockSpec((N, C), lambda i: (0, 0))],
        out_specs=pl.BlockSpec((C, N_PAD), lambda i: (0, 0)),
        out_shape=jax.ShapeDtypeStruct((C, N_PAD), jnp.float32),
    )(x)


def _sum_parts(p):
    return p[:, 0, :] + p[:, 1, :] + p[:, 2, :]


def _tc_mid(parts, W):
    def body(p_ref, w_ref, o_ref):
        agg = _sum_parts(p_ref[...])
        y = lax.dot_general(w_ref[...], agg, (((0,), (0,)), ((), ())),
                            preferred_element_type=jnp.float32)
        y = jnp.maximum(y, 0.0)
        s = jnp.sum(y * y, axis=0, keepdims=True)
        o_ref[...] = y * (1.0 / (jnp.sqrt(s) + EPS))

    return pl.pallas_call(
        body,
        grid=(N_PAD // BN,),
        in_specs=[
            pl.BlockSpec((C, CHUNKS, BN), lambda i: (0, 0, i)),
            pl.BlockSpec((C, C), lambda i: (0, 0)),
        ],
        out_specs=pl.BlockSpec((C, BN), lambda i: (0, i)),
        out_shape=jax.ShapeDtypeStruct((C, N_PAD), jnp.float32),
    )(parts, W)


def _tc_post(parts, W):
    def body(p_ref, w_ref, o_ref):
        agg = _sum_parts(p_ref[...])[:, :N]
        y = lax.dot_general(w_ref[...], agg, (((0,), (0,)), ((), ())),
                            preferred_element_type=jnp.float32)
        y = jnp.maximum(y, 0.0)
        m = jnp.max(y, axis=0, keepdims=True)
        e = jnp.exp(y - m)
        o_ref[...] = (e / jnp.sum(e, axis=0, keepdims=True)).T

    return pl.pallas_call(
        body,
        grid=(1,),
        in_specs=[
            pl.BlockSpec((C, CHUNKS, N_PAD), lambda i: (0, 0, 0)),
            pl.BlockSpec((C, C), lambda i: (0, 0)),
        ],
        out_specs=pl.BlockSpec((N, C), lambda i: (0, 0)),
        out_shape=jax.ShapeDtypeStruct((N, C), jnp.float32),
    )(parts, W)


def kernel(x, edge_index, W1, W2):
    src = edge_index[0]
    dst = edge_index[1]

    xn1 = _tc_pre(x)
    parts1 = _edge_pass(xn1, src, dst)
    xn2 = _tc_mid(parts1, W1)
    parts2 = _edge_pass(xn2, src, dst)
    return _tc_post(parts2, W2)


# speedup vs baseline: 1.1017x; 1.1017x over previous
import functools

import jax
import jax.numpy as jnp
from jax import lax
from jax.experimental import pallas as pl
from jax.experimental.pallas import tpu as pltpu
from jax.experimental.pallas import tpu_sc as plsc

N = 50000
C = 10
E = 3200000
EPS = 1e-15

N_PAD = 51200
BN = 51200
BE = 4096
CHUNKS = 3
CH = 1066672
NFULL = 260
NB16 = BE // 16


def _edge_pass(xn_t, src, dst):
    mesh = plsc.VectorSubcoreMesh(core_axis_name="c", subcore_axis_name="s")

    @functools.partial(
        pl.kernel,
        out_type=jax.ShapeDtypeStruct((C, CHUNKS, N_PAD), jnp.float32),
        mesh=mesh,
        compiler_params=pltpu.CompilerParams(needs_layout_passes=False),
        scratch_types=[
            pltpu.VMEM((N_PAD,), jnp.float32),
            pltpu.VMEM((N_PAD,), jnp.float32),
            pltpu.VMEM((BE,), jnp.int32),
            pltpu.VMEM((BE,), jnp.int32),
            pltpu.VMEM((BE,), jnp.int32),
            pltpu.VMEM((BE,), jnp.int32),
            pltpu.SemaphoreType.DMA,
            pltpu.SemaphoreType.DMA,
        ],
    )
    def k(xn_hbm, src_hbm, dst_hbm, out_hbm, table_v, acc_v, sb0, sb1,
          db0, db1, sem0, sem1):
        w = lax.axis_index("s") * 2 + lax.axis_index("c")

        @pl.when(w < C * CHUNKS)
        def _():
            cls = w // CHUNKS
            chunk = w % CHUNKS
            start = chunk * CH
            cnt = jnp.minimum(CH, E - start)
            tail16 = (cnt - NFULL * BE) // 16

            sbufs, dbufs, sems = (sb0, sb1), (db0, db1), (sem0, sem1)

            tcopy = pltpu.make_async_copy(xn_hbm.at[cls], table_v, sem0)
            tcopy.start()

            @plsc.parallel_loop(0, N_PAD // 16, unroll=4)
            def _(i):
                acc_v[pl.ds(i * 16, 16)] = jnp.zeros((16,), jnp.float32)

            tcopy.wait()

            def start_blk(b, slot):
                off = start + jnp.minimum(b * BE, cnt - BE)
                pltpu.async_copy(src_hbm.at[pl.ds(off, BE)], sbufs[slot],
                                 sems[slot])
                pltpu.async_copy(dst_hbm.at[pl.ds(off, BE)], dbufs[slot],
                                 sems[slot])

            def wait_blk(slot):
                pltpu.make_async_copy(src_hbm.at[pl.ds(0, BE)], sbufs[slot],
                                      sems[slot]).wait()
                pltpu.make_async_copy(dst_hbm.at[pl.ds(0, BE)], dbufs[slot],
                                      sems[slot]).wait()

            def group16(sb, db, i):
                s = sb[pl.ds(i * 16, 16)]
                d = db[pl.ds(i * 16, 16)]
                vals = plsc.load_gather(table_v, [s])
                plsc.addupdate_scatter(acc_v, [d], vals)

            start_blk(0, 0)
            start_blk(1, 1)

            def pair(g, _):
                for slot in (0, 1):
                    b = 2 * g + slot
                    wait_blk(slot)

                    @pl.when(b + 2 <= NFULL)
                    def _():
                        start_blk(b + 2, slot)

                    @plsc.parallel_loop(0, NB16, unroll=8)
                    def _(i):
                        group16(sbufs[slot], dbufs[slot], i)
                return 0

            lax.fori_loop(0, NFULL // 2, pair, 0)

            wait_blk(0)

            @plsc.parallel_loop(NB16 - tail16, NB16, unroll=2)
            def _(i):
                group16(sb0, db0, i)

            pltpu.sync_copy(acc_v, out_hbm.at[cls, chunk])

    return k(xn_t, src, dst)


def _tc_pre(x_t):
    def body(x_ref, o_ref):
        xb = x_ref[...]
        s = jnp.sum(xb * xb, axis=0, keepdims=True)
        o_ref[...] = xb * (1.0 / (jnp.sqrt(s) + EPS))

    return pl.pallas_call(
        body,
        grid=(1,),
        in_specs=[pl.BlockSpec((C, N_PAD), lambda i: (0, 0))],
        out_specs=pl.BlockSpec((C, N_PAD), lambda i: (0, 0)),
        out_shape=jax.ShapeDtypeStruct((C, N_PAD), jnp.float32),
    )(x_t)


def _sum_parts(p):
    return p[:, 0, :] + p[:, 1, :] + p[:, 2, :]


def _tc_mid(parts, W):
    def body(p_ref, w_ref, o_ref):
        agg = _sum_parts(p_ref[...])
        y = lax.dot_general(w_ref[...], agg, (((0,), (0,)), ((), ())),
                            preferred_element_type=jnp.float32)
        y = jnp.maximum(y, 0.0)
        s = jnp.sum(y * y, axis=0, keepdims=True)
        o_ref[...] = y * (1.0 / (jnp.sqrt(s) + EPS))

    return pl.pallas_call(
        body,
        grid=(N_PAD // BN,),
        in_specs=[
            pl.BlockSpec((C, CHUNKS, BN), lambda i: (0, 0, i)),
            pl.BlockSpec((C, C), lambda i: (0, 0)),
        ],
        out_specs=pl.BlockSpec((C, BN), lambda i: (0, i)),
        out_shape=jax.ShapeDtypeStruct((C, N_PAD), jnp.float32),
    )(parts, W)


def _tc_post(parts, W):
    def body(p_ref, w_ref, o_ref):
        agg = _sum_parts(p_ref[...])
        y = lax.dot_general(w_ref[...], agg, (((0,), (0,)), ((), ())),
                            preferred_element_type=jnp.float32)
        y = jnp.maximum(y, 0.0)
        m = jnp.max(y, axis=0, keepdims=True)
        e = jnp.exp(y - m)
        o_ref[...] = e / jnp.sum(e, axis=0, keepdims=True)

    return pl.pallas_call(
        body,
        grid=(1,),
        in_specs=[
            pl.BlockSpec((C, CHUNKS, N_PAD), lambda i: (0, 0, 0)),
            pl.BlockSpec((C, C), lambda i: (0, 0)),
        ],
        out_specs=pl.BlockSpec((C, N_PAD), lambda i: (0, 0)),
        out_shape=jax.ShapeDtypeStruct((C, N_PAD), jnp.float32),
    )(parts, W)


def kernel(x, edge_index, W1, W2):
    src = edge_index[0]
    dst = edge_index[1]
    x_t = jnp.pad(x.T, ((0, 0), (0, N_PAD - N)))

    xn1 = _tc_pre(x_t)
    parts1 = _edge_pass(xn1, src, dst)
    xn2 = _tc_mid(parts1, W1)
    parts2 = _edge_pass(xn2, src, dst)
    out_t = _tc_post(parts2, W2)
    return out_t[:, :N].T


# speedup vs baseline: 1.1677x; 1.0600x over previous
import functools

import jax
import jax.numpy as jnp
from jax import lax
from jax.experimental import pallas as pl
from jax.experimental.pallas import tpu as pltpu
from jax.experimental.pallas import tpu_sc as plsc

N = 50000
C = 10
E = 3200000
EPS = 1e-15

N_PAD = 51200
BN = 51200
BE = 8192
CHUNKS = 3
CH = 1066672
NFULL = 130
NB16 = BE // 16


def _edge_pass(xn_t, packed):
    mesh = plsc.VectorSubcoreMesh(core_axis_name="c", subcore_axis_name="s")

    @functools.partial(
        pl.kernel,
        out_type=jax.ShapeDtypeStruct((C, CHUNKS, N_PAD), jnp.float32),
        mesh=mesh,
        compiler_params=pltpu.CompilerParams(needs_layout_passes=False),
        scratch_types=[
            pltpu.VMEM((N_PAD,), jnp.float32),
            pltpu.VMEM((N_PAD,), jnp.float32),
            pltpu.VMEM((BE,), jnp.int32),
            pltpu.VMEM((BE,), jnp.int32),
            pltpu.SemaphoreType.DMA,
            pltpu.SemaphoreType.DMA,
        ],
    )
    def k(xn_hbm, pk_hbm, out_hbm, table_v, acc_v, pb0, pb1, sem0, sem1):
        w = lax.axis_index("s") * 2 + lax.axis_index("c")

        @pl.when(w < C * CHUNKS)
        def _():
            cls = w // CHUNKS
            chunk = w % CHUNKS
            start = chunk * CH
            cnt = jnp.minimum(CH, E - start)
            tail16 = (cnt - NFULL * BE) // 16

            pbufs, sems = (pb0, pb1), (sem0, sem1)

            tcopy = pltpu.make_async_copy(xn_hbm.at[cls], table_v, sem0)
            tcopy.start()

            @plsc.parallel_loop(0, N_PAD // 16, unroll=4)
            def _(i):
                acc_v[pl.ds(i * 16, 16)] = jnp.zeros((16,), jnp.float32)

            tcopy.wait()

            def start_blk(b, slot):
                off = start + jnp.minimum(b * BE, cnt - BE)
                pltpu.async_copy(pk_hbm.at[pl.ds(off, BE)], pbufs[slot],
                                 sems[slot])

            def wait_blk(slot):
                pltpu.make_async_copy(pk_hbm.at[pl.ds(0, BE)], pbufs[slot],
                                      sems[slot]).wait()

            def group16(pb, i):
                p = pb[pl.ds(i * 16, 16)]
                s = jnp.bitwise_and(p, 0xFFFF)
                d = jnp.bitwise_and(jnp.right_shift(p, 16), 0xFFFF)
                vals = plsc.load_gather(table_v, [s])
                plsc.addupdate_scatter(acc_v, [d], vals)

            start_blk(0, 0)
            start_blk(1, 1)

            def pair(g, _):
                for slot in (0, 1):
                    b = 2 * g + slot
                    wait_blk(slot)

                    @pl.when(b + 2 <= NFULL)
                    def _():
                        start_blk(b + 2, slot)

                    @plsc.parallel_loop(0, NB16, unroll=8)
                    def _(i):
                        group16(pbufs[slot], i)
                return 0

            lax.fori_loop(0, NFULL // 2, pair, 0)

            wait_blk(0)

            @plsc.parallel_loop(NB16 - tail16, NB16, unroll=2)
            def _(i):
                group16(pb0, i)

            pltpu.sync_copy(acc_v, out_hbm.at[cls, chunk])

    return k(xn_t, packed)


PK_R = 25
PK_W = E // PK_R
PK_BW = PK_W // 5


def _pack_edges(edge_index):
    ei = edge_index.reshape(2, PK_R, PK_W)

    def body(e_ref, o_ref):
        o_ref[...] = jnp.bitwise_or(e_ref[0], jnp.left_shift(e_ref[1], 16))

    packed = pl.pallas_call(
        body,
        grid=(PK_W // PK_BW,),
        in_specs=[pl.BlockSpec((2, PK_R, PK_BW), lambda i: (0, 0, i))],
        out_specs=pl.BlockSpec((PK_R, PK_BW), lambda i: (0, i)),
        out_shape=jax.ShapeDtypeStruct((PK_R, PK_W), jnp.int32),
    )(ei)
    return packed.reshape(E)


def _tc_pre(x_t):
    def body(x_ref, o_ref):
        xb = x_ref[...]
        s = jnp.sum(xb * xb, axis=0, keepdims=True)
        o_ref[...] = xb * (1.0 / (jnp.sqrt(s) + EPS))

    return pl.pallas_call(
        body,
        grid=(1,),
        in_specs=[pl.BlockSpec((C, N_PAD), lambda i: (0, 0))],
        out_specs=pl.BlockSpec((C, N_PAD), lambda i: (0, 0)),
        out_shape=jax.ShapeDtypeStruct((C, N_PAD), jnp.float32),
    )(x_t)


def _sum_parts(p):
    return p[:, 0, :] + p[:, 1, :] + p[:, 2, :]


def _tc_mid(parts, W):
    def body(p_ref, w_ref, o_ref):
        agg = _sum_parts(p_ref[...])
        y = lax.dot_general(w_ref[...], agg, (((0,), (0,)), ((), ())),
                            preferred_element_type=jnp.float32)
        y = jnp.maximum(y, 0.0)
        s = jnp.sum(y * y, axis=0, keepdims=True)
        o_ref[...] = y * (1.0 / (jnp.sqrt(s) + EPS))

    return pl.pallas_call(
        body,
        grid=(N_PAD // BN,),
        in_specs=[
            pl.BlockSpec((C, CHUNKS, BN), lambda i: (0, 0, i)),
            pl.BlockSpec((C, C), lambda i: (0, 0)),
        ],
        out_specs=pl.BlockSpec((C, BN), lambda i: (0, i)),
        out_shape=jax.ShapeDtypeStruct((C, N_PAD), jnp.float32),
    )(parts, W)


def _tc_post(parts, W):
    def body(p_ref, w_ref, o_ref):
        agg = _sum_parts(p_ref[...])
        y = lax.dot_general(w_ref[...], agg, (((0,), (0,)), ((), ())),
                            preferred_element_type=jnp.float32)
        y = jnp.maximum(y, 0.0)
        m = jnp.max(y, axis=0, keepdims=True)
        e = jnp.exp(y - m)
        o_ref[...] = e / jnp.sum(e, axis=0, keepdims=True)

    return pl.pallas_call(
        body,
        grid=(1,),
        in_specs=[
            pl.BlockSpec((C, CHUNKS, N_PAD), lambda i: (0, 0, 0)),
            pl.BlockSpec((C, C), lambda i: (0, 0)),
        ],
        out_specs=pl.BlockSpec((C, N_PAD), lambda i: (0, 0)),
        out_shape=jax.ShapeDtypeStruct((C, N_PAD), jnp.float32),
    )(parts, W)


def kernel(x, edge_index, W1, W2):
    packed = _pack_edges(edge_index)
    x_t = jnp.pad(x.T, ((0, 0), (0, N_PAD - N)))

    xn1 = _tc_pre(x_t)
    parts1 = _edge_pass(xn1, packed)
    xn2 = _tc_mid(parts1, W1)
    parts2 = _edge_pass(xn2, packed)
    out_t = _tc_post(parts2, W2)
    return out_t[:, :N].T


# speedup vs baseline: 1.3381x; 1.1459x over previous
import functools

import jax
import jax.numpy as jnp
from jax import lax
from jax.experimental import pallas as pl
from jax.experimental.pallas import tpu as pltpu
from jax.experimental.pallas import tpu_sc as plsc

N = 50000
C = 10
E = 3200000
EPS = 1e-15

N_PAD = 51200
BN = 51200
BE = 8192
CHUNKS = 3
CH = 1066672
NFULL = 130
NB16 = BE // 16


def _edge_pass(xn_t, packed):
    mesh = plsc.VectorSubcoreMesh(core_axis_name="c", subcore_axis_name="s")

    @functools.partial(
        pl.kernel,
        out_type=jax.ShapeDtypeStruct((C, CHUNKS, N_PAD), jnp.float32),
        mesh=mesh,
        compiler_params=pltpu.CompilerParams(needs_layout_passes=False),
        scratch_types=[
            pltpu.VMEM((N_PAD,), jnp.float32),
            pltpu.VMEM((N_PAD,), jnp.float32),
            pltpu.VMEM((BE,), jnp.int32),
            pltpu.VMEM((BE,), jnp.int32),
            pltpu.SemaphoreType.DMA,
            pltpu.SemaphoreType.DMA,
        ],
    )
    def k(xn_hbm, pk_hbm, out_hbm, table_v, acc_v, pb0, pb1, sem0, sem1):
        w = lax.axis_index("s") * 2 + lax.axis_index("c")

        @pl.when(w < C * CHUNKS)
        def _():
            cls = w // CHUNKS
            chunk = w % CHUNKS
            start = chunk * CH
            cnt = jnp.minimum(CH, E - start)
            tail16 = (cnt - NFULL * BE) // 16

            pbufs, sems = (pb0, pb1), (sem0, sem1)

            tcopy = pltpu.make_async_copy(xn_hbm.at[cls], table_v, sem0)
            tcopy.start()

            @plsc.parallel_loop(0, N_PAD // 16, unroll=4)
            def _(i):
                acc_v[pl.ds(i * 16, 16)] = jnp.zeros((16,), jnp.float32)

            tcopy.wait()

            def start_blk(b, slot):
                off = start + jnp.minimum(b * BE, cnt - BE)
                pltpu.async_copy(pk_hbm.at[pl.ds(off, BE)], pbufs[slot],
                                 sems[slot])

            def wait_blk(slot):
                pltpu.make_async_copy(pk_hbm.at[pl.ds(0, BE)], pbufs[slot],
                                      sems[slot]).wait()

            def group16(pb, i):
                p = pb[pl.ds(i * 16, 16)]
                s = jnp.bitwise_and(p, 0xFFFF)
                d = jnp.bitwise_and(jnp.right_shift(p, 16), 0xFFFF)
                vals = plsc.load_gather(table_v, [s])
                plsc.addupdate_scatter(acc_v, [d], vals)

            start_blk(0, 0)
            start_blk(1, 1)

            def pair(g, _):
                for slot in (0, 1):
                    b = 2 * g + slot
                    wait_blk(slot)

                    @pl.when(b + 2 <= NFULL)
                    def _():
                        start_blk(b + 2, slot)

                    @plsc.parallel_loop(0, NB16, unroll=8)
                    def _(i):
                        group16(pbufs[slot], i)
                return 0

            lax.fori_loop(0, NFULL // 2, pair, 0)

            wait_blk(0)

            @plsc.parallel_loop(NB16 - tail16, NB16, unroll=2)
            def _(i):
                group16(pb0, i)

            pltpu.sync_copy(acc_v, out_hbm.at[cls, chunk])

    return k(xn_t, packed)


PK_R = 25
PK_W = E // PK_R
PK_BW = PK_W // 5


def _pack_edges(edge_index):
    ei = edge_index.reshape(2, PK_R, PK_W)

    def body(e_ref, o_ref):
        o_ref[...] = jnp.bitwise_or(e_ref[0], jnp.left_shift(e_ref[1], 16))

    packed = pl.pallas_call(
        body,
        grid=(PK_W // PK_BW,),
        in_specs=[pl.BlockSpec((2, PK_R, PK_BW), lambda i: (0, 0, i))],
        out_specs=pl.BlockSpec((PK_R, PK_BW), lambda i: (0, i)),
        out_shape=jax.ShapeDtypeStruct((PK_R, PK_W), jnp.int32),
    )(ei)
    return packed.reshape(E)


def _tc_pre(x_t):
    def body(x_ref, o_ref):
        xb = x_ref[...]
        s = jnp.sum(xb * xb, axis=0, keepdims=True)
        o_ref[...] = xb * (1.0 / (jnp.sqrt(s) + EPS))

    return pl.pallas_call(
        body,
        grid=(1,),
        in_specs=[pl.BlockSpec((C, N_PAD), lambda i: (0, 0))],
        out_specs=pl.BlockSpec((C, N_PAD), lambda i: (0, 0)),
        out_shape=jax.ShapeDtypeStruct((C, N_PAD), jnp.float32),
    )(x_t)


def _sum_parts(p):
    return p[:, 0, :] + p[:, 1, :] + p[:, 2, :]


def _tc_mid(parts, W):
    def body(p_ref, w_ref, o_ref):
        agg = _sum_parts(p_ref[...])
        y = lax.dot_general(w_ref[...], agg, (((0,), (0,)), ((), ())),
                            preferred_element_type=jnp.float32)
        y = jnp.maximum(y, 0.0)
        s = jnp.sum(y * y, axis=0, keepdims=True)
        o_ref[...] = y * (1.0 / (jnp.sqrt(s) + EPS))

    return pl.pallas_call(
        body,
        grid=(N_PAD // BN,),
        in_specs=[
            pl.BlockSpec((C, CHUNKS, BN), lambda i: (0, 0, i)),
            pl.BlockSpec((C, C), lambda i: (0, 0)),
        ],
        out_specs=pl.BlockSpec((C, BN), lambda i: (0, i)),
        out_shape=jax.ShapeDtypeStruct((C, N_PAD), jnp.float32),
    )(parts, W)


def _tc_post(parts, W):
    def body(p_ref, w_ref, o_ref):
        agg = _sum_parts(p_ref[...])
        y = lax.dot_general(w_ref[...], agg, (((0,), (0,)), ((), ())),
                            preferred_element_type=jnp.float32)
        y = jnp.maximum(y, 0.0)
        m = jnp.max(y, axis=0, keepdims=True)
        e = jnp.exp(y - m)
        o_ref[...] = e / jnp.sum(e, axis=0, keepdims=True)

    return pl.pallas_call(
        body,
        grid=(1,),
        in_specs=[
            pl.BlockSpec((C, CHUNKS, N_PAD), lambda i: (0, 0, 0)),
            pl.BlockSpec((C, C), lambda i: (0, 0)),
        ],
        out_specs=pl.BlockSpec((C, N_PAD), lambda i: (0, 0)),
        out_shape=jax.ShapeDtypeStruct((C, N_PAD), jnp.float32),
    )(parts, W)


def kernel(x, edge_index, W1, W2):
    packed = jnp.bitwise_or(edge_index[0], jnp.left_shift(edge_index[1], 16))
    x_t = jnp.pad(x.T, ((0, 0), (0, N_PAD - N)))

    xn1 = _tc_pre(x_t)
    parts1 = _edge_pass(xn1, packed)
    xn2 = _tc_mid(parts1, W1)
    parts2 = _edge_pass(xn2, packed)
    out_t = _tc_post(parts2, W2)
    return out_t[:, :N].T


# speedup vs baseline: 1.3519x; 1.0103x over previous
import functools

import jax
import jax.numpy as jnp
from jax import lax
from jax.experimental import pallas as pl
from jax.experimental.pallas import tpu as pltpu
from jax.experimental.pallas import tpu_sc as plsc

N = 50000
C = 10
E = 3200000
EPS = 1e-15

N_PAD = 51200
BN = 51200
BE = 8192
CHUNKS = 3
CH = 1066672
NFULL = 130
NB16 = BE // 16


def _edge_pass(xn_t, packed):
    mesh = plsc.VectorSubcoreMesh(core_axis_name="c", subcore_axis_name="s")

    @functools.partial(
        pl.kernel,
        out_type=jax.ShapeDtypeStruct((C, CHUNKS, N_PAD), jnp.float32),
        mesh=mesh,
        compiler_params=pltpu.CompilerParams(needs_layout_passes=False),
        scratch_types=[
            pltpu.VMEM((N_PAD,), jnp.float32),
            pltpu.VMEM((N_PAD,), jnp.float32),
            pltpu.VMEM((BE,), jnp.int32),
            pltpu.VMEM((BE,), jnp.int32),
            pltpu.SemaphoreType.DMA,
            pltpu.SemaphoreType.DMA,
        ],
    )
    def k(xn_hbm, pk_hbm, out_hbm, table_v, acc_v, pb0, pb1, sem0, sem1):
        w = lax.axis_index("s") * 2 + lax.axis_index("c")

        @pl.when(w < C * CHUNKS)
        def _():
            cls = w // CHUNKS
            chunk = w % CHUNKS
            start = chunk * CH
            cnt = jnp.minimum(CH, E - start)
            tail16 = (cnt - NFULL * BE) // 16

            pbufs, sems = (pb0, pb1), (sem0, sem1)

            tcopy = pltpu.make_async_copy(xn_hbm.at[cls], table_v, sem0)
            tcopy.start()

            @plsc.parallel_loop(0, N_PAD // 16, unroll=4)
            def _(i):
                acc_v[pl.ds(i * 16, 16)] = jnp.zeros((16,), jnp.float32)

            tcopy.wait()

            def start_blk(b, slot):
                off = start + jnp.minimum(b * BE, cnt - BE)
                pltpu.async_copy(pk_hbm.at[pl.ds(off, BE)], pbufs[slot],
                                 sems[slot])

            def wait_blk(slot):
                pltpu.make_async_copy(pk_hbm.at[pl.ds(0, BE)], pbufs[slot],
                                      sems[slot]).wait()

            def group16(pb, i):
                p = pb[pl.ds(i * 16, 16)]
                s = jnp.bitwise_and(p, 0xFFFF)
                d = jnp.bitwise_and(jnp.right_shift(p, 16), 0xFFFF)
                vals = plsc.load_gather(table_v, [s])
                plsc.addupdate_scatter(acc_v, [d], vals)

            start_blk(0, 0)
            start_blk(1, 1)

            def pair(g, _):
                for slot in (0, 1):
                    b = 2 * g + slot
                    wait_blk(slot)

                    @pl.when(b + 2 <= NFULL)
                    def _():
                        start_blk(b + 2, slot)

                    @plsc.parallel_loop(0, NB16, unroll=16)
                    def _(i):
                        group16(pbufs[slot], i)
                return 0

            lax.fori_loop(0, NFULL // 2, pair, 0)

            wait_blk(0)

            @plsc.parallel_loop(NB16 - tail16, NB16, unroll=2)
            def _(i):
                group16(pb0, i)

            pltpu.sync_copy(acc_v, out_hbm.at[cls, chunk])

    return k(xn_t, packed)


PK_R = 25
PK_W = E // PK_R
PK_BW = PK_W // 5


def _pack_edges(edge_index):
    ei = edge_index.reshape(2, PK_R, PK_W)

    def body(e_ref, o_ref):
        o_ref[...] = jnp.bitwise_or(e_ref[0], jnp.left_shift(e_ref[1], 16))

    packed = pl.pallas_call(
        body,
        grid=(PK_W // PK_BW,),
        in_specs=[pl.BlockSpec((2, PK_R, PK_BW), lambda i: (0, 0, i))],
        out_specs=pl.BlockSpec((PK_R, PK_BW), lambda i: (0, i)),
        out_shape=jax.ShapeDtypeStruct((PK_R, PK_W), jnp.int32),
    )(ei)
    return packed.reshape(E)


def _tc_pre(x_t):
    def body(x_ref, o_ref):
        xb = x_ref[...]
        s = jnp.sum(xb * xb, axis=0, keepdims=True)
        o_ref[...] = xb * (1.0 / (jnp.sqrt(s) + EPS))

    return pl.pallas_call(
        body,
        grid=(1,),
        in_specs=[pl.BlockSpec((C, N_PAD), lambda i: (0, 0))],
        out_specs=pl.BlockSpec((C, N_PAD), lambda i: (0, 0)),
        out_shape=jax.ShapeDtypeStruct((C, N_PAD), jnp.float32),
    )(x_t)


def _sum_parts(p):
    return p[:, 0, :] + p[:, 1, :] + p[:, 2, :]


def _tc_mid(parts, W):
    def body(p_ref, w_ref, o_ref):
        agg = _sum_parts(p_ref[...])
        y = lax.dot_general(w_ref[...], agg, (((0,), (0,)), ((), ())),
                            preferred_element_type=jnp.float32)
        y = jnp.maximum(y, 0.0)
        s = jnp.sum(y * y, axis=0, keepdims=True)
        o_ref[...] = y * (1.0 / (jnp.sqrt(s) + EPS))

    return pl.pallas_call(
        body,
        grid=(N_PAD // BN,),
        in_specs=[
            pl.BlockSpec((C, CHUNKS, BN), lambda i: (0, 0, i)),
            pl.BlockSpec((C, C), lambda i: (0, 0)),
        ],
        out_specs=pl.BlockSpec((C, BN), lambda i: (0, i)),
        out_shape=jax.ShapeDtypeStruct((C, N_PAD), jnp.float32),
    )(parts, W)


def _tc_post(parts, W):
    def body(p_ref, w_ref, o_ref):
        agg = _sum_parts(p_ref[...])
        y = lax.dot_general(w_ref[...], agg, (((0,), (0,)), ((), ())),
                            preferred_element_type=jnp.float32)
        y = jnp.maximum(y, 0.0)
        m = jnp.max(y, axis=0, keepdims=True)
        e = jnp.exp(y - m)
        o_ref[...] = e / jnp.sum(e, axis=0, keepdims=True)

    return pl.pallas_call(
        body,
        grid=(1,),
        in_specs=[
            pl.BlockSpec((C, CHUNKS, N_PAD), lambda i: (0, 0, 0)),
            pl.BlockSpec((C, C), lambda i: (0, 0)),
        ],
        out_specs=pl.BlockSpec((C, N_PAD), lambda i: (0, 0)),
        out_shape=jax.ShapeDtypeStruct((C, N_PAD), jnp.float32),
    )(parts, W)


def kernel(x, edge_index, W1, W2):
    packed = jnp.bitwise_or(edge_index[0], jnp.left_shift(edge_index[1], 16))
    x_t = jnp.pad(x.T, ((0, 0), (0, N_PAD - N)))

    xn1 = _tc_pre(x_t)
    parts1 = _edge_pass(xn1, packed)
    xn2 = _tc_mid(parts1, W1)
    parts2 = _edge_pass(xn2, packed)
    out_t = _tc_post(parts2, W2)
    return out_t[:, :N].T
